# baseline (speedup 1.0000x reference)
import functools

import numpy as np
import jax
import jax.numpy as jnp
from jax import lax
from jax.experimental import pallas as pl
from jax.experimental.pallas import tpu as pltpu
from jax.experimental.pallas import tpu_sc as plsc

_BOUND = np.array([[-10.0, 10.0], [-10.0, 10.0], [-5.0, 5.0]], dtype=np.float32)
_L = 16
_F = 2
_T = 2 ** 19
_B_SCALE = 1.3819
_VOXEL = 1.25
_MAX_DIS = float(np.ceil((_BOUND[:, 1] - _BOUND[:, 0]).max()))
_N_MIN = int(_MAX_DIS / _VOXEL)
_RES = [int(np.floor(_N_MIN * (_B_SCALE ** l))) for l in range(_L)]
_DENSE = [(r + 1) ** 3 <= _T for r in _RES]
_P2 = np.uint32(2654435761)
_P3 = np.uint32(805459861)

_NW = 32
_C = 32
_R = _L * 8


def _idx_wgt_body(xn_ref, idx_ref, wgt_ref):
    x = xn_ref[0:1, :]
    y = xn_ref[1:2, :]
    z = xn_ref[2:3, :]
    for l in range(_L):
        res = _RES[l]
        s = res + 1
        px = x * res
        py = y * res
        pz = z * res
        fx = jnp.floor(px)
        fy = jnp.floor(py)
        fz = jnp.floor(pz)
        wx = px - fx
        wy = py - fy
        wz = pz - fz
        ix = fx.astype(jnp.int32)
        iy = fy.astype(jnp.int32)
        iz = fz.astype(jnp.int32)
        for k in range(8):
            ox, oy, oz = k & 1, (k >> 1) & 1, (k >> 2) & 1
            cx = ix + ox
            cy = iy + oy
            cz = iz + oz
            if _DENSE[l]:
                idx = cx + cy * s + cz * (s * s)
            else:
                h = (cx.astype(jnp.uint32)
                     ^ (cy.astype(jnp.uint32) * _P2)
                     ^ (cz.astype(jnp.uint32) * _P3))
                idx = (h & jnp.uint32(_T - 1)).astype(jnp.int32)
            idx = idx + l * _T
            w = ((wx if ox else 1.0 - wx)
                 * (wy if oy else 1.0 - wy)
                 * (wz if oz else 1.0 - wz))
            r = l * 8 + k
            idx_ref[r:r + 1, :] = idx
            wgt_ref[r:r + 1, :] = w


def _sc_encode_body(tbl_s, tbl_r, idxh, wgth, outh, idx_v, sidx_v, wgt_v,
                    rows_v, feat_v, sem):
    n = idxh.shape[1]
    pw = n // _NW
    nch = pw // _C
    wid = lax.axis_index("s") * 2 + lax.axis_index("c")
    iota = lax.iota(jnp.int32, 16)

    def chunk_body(j, carry):
        base = wid * pw + j * _C
        pltpu.sync_copy(idxh.at[:, pl.ds(base, _C)], idx_v)
        pltpu.sync_copy(wgth.at[:, pl.ds(base, _C)], wgt_v)

        def shift_body(r, c1):
            def shift_grp(g2, c2):
                iv = idx_v[r, pl.ds(g2 * 16, 16)]
                sidx_v[pl.ds(r * _C + g2 * 16, 16)] = lax.shift_right_logical(
                    iv, 3)
                return c2
            lax.fori_loop(0, _C // 16, shift_grp, 0)
            return c1
        lax.fori_loop(0, _R, shift_body, 0)

        for tbl, foff in ((tbl_s, 0), (tbl_r, 32)):
            copies = [
                pltpu.async_copy(tbl.at[sidx_v.at[pl.ds(r * _C, _C)]],
                                 rows_v.at[pl.ds(r * _C, _C)], sem)
                for r in range(_R)
            ]
            for cp in copies:
                cp.wait()

            def lvl_body(l, c3):
                def grp_body(g, c4):
                    acc = [jnp.zeros((16,), jnp.float32) for _ in range(2)]
                    for k in range(8):
                        r = l * 8 + k
                        w = wgt_v[r, pl.ds(g * 16, 16)]
                        iv = idx_v[r, pl.ds(g * 16, 16)]
                        colbase = (iv & 7) * 2
                        ridx = r * _C + g * 16 + iota
                        for c in range(2):
                            vals = plsc.load_gather(rows_v,
                                                    [ridx, colbase + c])
                            acc[c] = acc[c] + vals * w
                    for c in range(2):
                        feat_v[foff + l * 2 + c, pl.ds(g * 16, 16)] = acc[c]
                    return c4
                lax.fori_loop(0, _C // 16, grp_body, 0)
                return c3
            lax.fori_loop(0, _L, lvl_body, 0)
        pltpu.sync_copy(feat_v, outh.at[:, pl.ds(base, _C)])
        return carry

    lax.fori_loop(0, nch, chunk_body, 0)


@functools.lru_cache(maxsize=2)
def _make_sc_encode(n):
    mesh = plsc.VectorSubcoreMesh(core_axis_name="c", subcore_axis_name="s")
    return functools.partial(
        pl.kernel,
        out_type=jax.ShapeDtypeStruct((64, n), jnp.float32),
        mesh=mesh,
        compiler_params=pltpu.CompilerParams(
            needs_layout_passes=False, use_tc_tiling_on_sc=False),
        scratch_types=[
            pltpu.VMEM((_R, _C), jnp.int32),
            pltpu.VMEM((_R * _C,), jnp.int32),
            pltpu.VMEM((_R, _C), jnp.float32),
            pltpu.VMEM((_R * _C, 16), jnp.float32),
            pltpu.VMEM((64, _C), jnp.float32),
            pltpu.SemaphoreType.DMA,
        ],
    )(_sc_encode_body)


def _mlp_body(f_ref, ws1, ws2, wc1, wc2, wc3, rgb_ref, sdf_ref):
    dn = (((0,), (0,)), ((), ()))
    prec = lax.Precision.HIGHEST
    fs = f_ref[0:32, :]
    fr = f_ref[32:64, :]
    h = jnp.maximum(lax.dot_general(ws1[...], fs, dn, precision=prec), 0.0)
    sdf_ref[...] = lax.dot_general(ws2[...], h, dn, precision=prec)
    h1 = jnp.maximum(lax.dot_general(wc1[...], fr, dn, precision=prec), 0.0)
    h2 = jnp.maximum(lax.dot_general(wc2[...], h1, dn, precision=prec), 0.0)
    rgb_ref[...] = jax.nn.sigmoid(
        lax.dot_general(wc3[...], h2, dn, precision=prec))


def kernel(xyz, table_sdf, table_rgb, Ws1, Ws2, Wc1, Wc2, Wc3):
    n = xyz.shape[0]
    bmin = jnp.asarray(_BOUND[:, 0])
    bdis = jnp.asarray(_BOUND[:, 1] - _BOUND[:, 0])
    xn_t = ((xyz - bmin) / bdis).T

    bn = 512
    idx_t, wgt_t = pl.pallas_call(
        _idx_wgt_body,
        grid=(n // bn,),
        in_specs=[pl.BlockSpec((3, bn), lambda i: (0, i))],
        out_specs=[pl.BlockSpec((_R, bn), lambda i: (0, i)),
                   pl.BlockSpec((_R, bn), lambda i: (0, i))],
        out_shape=[jax.ShapeDtypeStruct((_R, n), jnp.int32),
                   jax.ShapeDtypeStruct((_R, n), jnp.float32)],
    )(xn_t)

    tbl_s = table_sdf.reshape(_L * _T * _F // 16, 16)
    tbl_r = table_rgb.reshape(_L * _T * _F // 16, 16)
    feats = _make_sc_encode(n)(tbl_s, tbl_r, idx_t, wgt_t)

    bm = 4096
    rgb_t, sdf_t = pl.pallas_call(
        _mlp_body,
        grid=(n // bm,),
        in_specs=[pl.BlockSpec((64, bm), lambda i: (0, i)),
                  pl.BlockSpec(Ws1.shape, lambda i: (0, 0)),
                  pl.BlockSpec(Ws2.shape, lambda i: (0, 0)),
                  pl.BlockSpec(Wc1.shape, lambda i: (0, 0)),
                  pl.BlockSpec(Wc2.shape, lambda i: (0, 0)),
                  pl.BlockSpec(Wc3.shape, lambda i: (0, 0))],
        out_specs=[pl.BlockSpec((3, bm), lambda i: (0, i)),
                   pl.BlockSpec((1, bm), lambda i: (0, i))],
        out_shape=[jax.ShapeDtypeStruct((3, n), jnp.float32),
                   jax.ShapeDtypeStruct((1, n), jnp.float32)],
    )(feats, Ws1, Ws2, Wc1, Wc2, Wc3)

    return (rgb_t.T, sdf_t[0])


# speedup vs baseline: 4.9807x; 4.9807x over previous
import functools

import numpy as np
import jax
import jax.numpy as jnp
from jax import lax
from jax.experimental import pallas as pl
from jax.experimental.pallas import tpu as pltpu
from jax.experimental.pallas import tpu_sc as plsc

_BOUND = np.array([[-10.0, 10.0], [-10.0, 10.0], [-5.0, 5.0]], dtype=np.float32)
_L = 16
_F = 2
_T = 2 ** 19
_B_SCALE = 1.3819
_VOXEL = 1.25
_MAX_DIS = float(np.ceil((_BOUND[:, 1] - _BOUND[:, 0]).max()))
_N_MIN = int(_MAX_DIS / _VOXEL)
_RES = [int(np.floor(_N_MIN * (_B_SCALE ** l))) for l in range(_L)]
_DENSE = [(r + 1) ** 3 <= _T for r in _RES]
_P2 = np.uint32(2654435761)
_P3 = np.uint32(805459861)

_NW = 32
_C = 32
_R = _L * 8


def _idx_wgt_body(xn_ref, idx_ref, wgt_ref):
    x = xn_ref[0:1, :]
    y = xn_ref[1:2, :]
    z = xn_ref[2:3, :]
    for l in range(_L):
        res = _RES[l]
        s = res + 1
        px = x * res
        py = y * res
        pz = z * res
        fx = jnp.floor(px)
        fy = jnp.floor(py)
        fz = jnp.floor(pz)
        wx = px - fx
        wy = py - fy
        wz = pz - fz
        ix = fx.astype(jnp.int32)
        iy = fy.astype(jnp.int32)
        iz = fz.astype(jnp.int32)
        for k in range(8):
            ox, oy, oz = k & 1, (k >> 1) & 1, (k >> 2) & 1
            cx = ix + ox
            cy = iy + oy
            cz = iz + oz
            if _DENSE[l]:
                idx = cx + cy * s + cz * (s * s)
            else:
                h = (cx.astype(jnp.uint32)
                     ^ (cy.astype(jnp.uint32) * _P2)
                     ^ (cz.astype(jnp.uint32) * _P3))
                idx = (h & jnp.uint32(_T - 1)).astype(jnp.int32)
            idx = idx + l * _T
            w = ((wx if ox else 1.0 - wx)
                 * (wy if oy else 1.0 - wy)
                 * (wz if oz else 1.0 - wz))
            r = l * 8 + k
            idx_ref[r:r + 1, :] = idx
            wgt_ref[r:r + 1, :] = w


def _interleave_body(s_ref, r_ref, os_ref, or_ref):
    for ref, oref in ((s_ref, os_ref), (r_ref, or_ref)):
        x = ref[...]
        br = x.shape[0]
        y = x.reshape(br, 2, 16, 8).transpose(0, 2, 1, 3).reshape(br, 256)
        oref[...] = y


def _sc_encode_body(tbl_s, tbl_r, idxh, wgth, outh, idx_v, sidx_v, wgt_v,
                    rows_v, feat_v, sem):
    n = idxh.shape[1]
    pw = n // _NW
    nch = pw // _C
    wid = lax.axis_index("s") * 2 + lax.axis_index("c")
    iota = lax.iota(jnp.int32, 16)

    def chunk_body(j, carry):
        base = wid * pw + j * _C
        pltpu.sync_copy(idxh.at[:, pl.ds(base, _C)], idx_v)
        pltpu.sync_copy(wgth.at[:, pl.ds(base, _C)], wgt_v)

        def shift_body(r, c1):
            def shift_grp(g2, c2):
                iv = idx_v[r, pl.ds(g2 * 16, 16)]
                sidx_v[pl.ds(r * _C + g2 * 16, 16)] = lax.shift_right_logical(
                    iv, 3)
                return c2
            lax.fori_loop(0, _C // 16, shift_grp, 0)
            return c1
        lax.fori_loop(0, _R, shift_body, 0)

        for tbl, foff in ((tbl_s, 0), (tbl_r, 32)):
            copies = [
                pltpu.async_copy(tbl.at[sidx_v.at[pl.ds(r * _C, _C)]],
                                 rows_v.at[pl.ds(r * _C, _C)], sem)
                for r in range(_R)
            ]
            for cp in copies:
                cp.wait()

            def lvl_body(l, c3):
                def grp_body(g, c4):
                    acc = [jnp.zeros((16,), jnp.float32) for _ in range(2)]
                    for k in range(8):
                        r = l * 8 + k
                        w = wgt_v[r, pl.ds(g * 16, 16)]
                        iv = idx_v[r, pl.ds(g * 16, 16)]
                        colbase = iv & 7
                        ridx = r * _C + g * 16 + iota
                        for c in range(2):
                            vals = plsc.load_gather(rows_v,
                                                    [ridx, colbase + c * 8])
                            acc[c] = acc[c] + vals * w
                    for c in range(2):
                        feat_v[foff + l * 2 + c, pl.ds(g * 16, 16)] = acc[c]
                    return c4
                lax.fori_loop(0, _C // 16, grp_body, 0)
                return c3
            lax.fori_loop(0, _L, lvl_body, 0)
        pltpu.sync_copy(feat_v, outh.at[:, pl.ds(base, _C)])
        return carry

    lax.fori_loop(0, nch, chunk_body, 0)


@functools.lru_cache(maxsize=2)
def _make_sc_encode(n):
    mesh = plsc.VectorSubcoreMesh(core_axis_name="c", subcore_axis_name="s")
    return functools.partial(
        pl.kernel,
        out_type=jax.ShapeDtypeStruct((64, n), jnp.float32),
        mesh=mesh,
        compiler_params=pltpu.CompilerParams(
            needs_layout_passes=False, use_tc_tiling_on_sc=False),
        scratch_types=[
            pltpu.VMEM((_R, _C), jnp.int32),
            pltpu.VMEM((_R * _C,), jnp.int32),
            pltpu.VMEM((_R, _C), jnp.float32),
            pltpu.VMEM((_R * _C, 16), jnp.float32),
            pltpu.VMEM((64, _C), jnp.float32),
            pltpu.SemaphoreType.DMA,
        ],
    )(_sc_encode_body)


def _mlp_body(f_ref, ws1, ws2, wc1, wc2, wc3, rgb_ref, sdf_ref):
    dn = (((0,), (0,)), ((), ()))
    prec = lax.Precision.HIGHEST
    fs = f_ref[0:32, :]
    fr = f_ref[32:64, :]
    h = jnp.maximum(lax.dot_general(ws1[...], fs, dn, precision=prec), 0.0)
    sdf_ref[...] = lax.dot_general(ws2[...], h, dn, precision=prec)
    h1 = jnp.maximum(lax.dot_general(wc1[...], fr, dn, precision=prec), 0.0)
    h2 = jnp.maximum(lax.dot_general(wc2[...], h1, dn, precision=prec), 0.0)
    rgb_ref[...] = jax.nn.sigmoid(
        lax.dot_general(wc3[...], h2, dn, precision=prec))


def kernel(xyz, table_sdf, table_rgb, Ws1, Ws2, Wc1, Wc2, Wc3):
    n = xyz.shape[0]
    bmin = jnp.asarray(_BOUND[:, 0])
    bdis = jnp.asarray(_BOUND[:, 1] - _BOUND[:, 0])
    xn_t = ((xyz - bmin) / bdis).T

    bn = 512
    idx_t, wgt_t = pl.pallas_call(
        _idx_wgt_body,
        grid=(n // bn,),
        in_specs=[pl.BlockSpec((3, bn), lambda i: (0, i))],
        out_specs=[pl.BlockSpec((_R, bn), lambda i: (0, i)),
                   pl.BlockSpec((_R, bn), lambda i: (0, i))],
        out_shape=[jax.ShapeDtypeStruct((_R, n), jnp.int32),
                   jax.ShapeDtypeStruct((_R, n), jnp.float32)],
    )(xn_t)

    nb = _T // 128
    tp_s = (table_sdf.reshape(_L, nb, 128, _F).transpose(0, 1, 3, 2)
            .reshape(_L * nb, 256))
    tp_r = (table_rgb.reshape(_L, nb, 128, _F).transpose(0, 1, 3, 2)
            .reshape(_L * nb, 256))
    br = 256
    il_s, il_r = pl.pallas_call(
        _interleave_body,
        grid=(_L * nb // br,),
        in_specs=[pl.BlockSpec((br, 256), lambda i: (i, 0)),
                  pl.BlockSpec((br, 256), lambda i: (i, 0))],
        out_specs=[pl.BlockSpec((br, 256), lambda i: (i, 0)),
                   pl.BlockSpec((br, 256), lambda i: (i, 0))],
        out_shape=[jax.ShapeDtypeStruct((_L * nb, 256), jnp.float32),
                   jax.ShapeDtypeStruct((_L * nb, 256), jnp.float32)],
    )(tp_s, tp_r)
    tbl_s = il_s.reshape(_L * _T // 8, 16)
    tbl_r = il_r.reshape(_L * _T // 8, 16)
    feats = _make_sc_encode(n)(tbl_s, tbl_r, idx_t, wgt_t)

    bm = 4096
    rgb_t, sdf_t = pl.pallas_call(
        _mlp_body,
        grid=(n // bm,),
        in_specs=[pl.BlockSpec((64, bm), lambda i: (0, i)),
                  pl.BlockSpec(Ws1.shape, lambda i: (0, 0)),
                  pl.BlockSpec(Ws2.shape, lambda i: (0, 0)),
                  pl.BlockSpec(Wc1.shape, lambda i: (0, 0)),
                  pl.BlockSpec(Wc2.shape, lambda i: (0, 0)),
                  pl.BlockSpec(Wc3.shape, lambda i: (0, 0))],
        out_specs=[pl.BlockSpec((3, bm), lambda i: (0, i)),
                   pl.BlockSpec((1, bm), lambda i: (0, i))],
        out_shape=[jax.ShapeDtypeStruct((3, n), jnp.float32),
                   jax.ShapeDtypeStruct((1, n), jnp.float32)],
    )(feats, Ws1, Ws2, Wc1, Wc2, Wc3)

    return (rgb_t.T, sdf_t[0])


# speedup vs baseline: 7.3027x; 1.4662x over previous
import functools

import numpy as np
import jax
import jax.numpy as jnp
from jax import lax
from jax.experimental import pallas as pl
from jax.experimental.pallas import tpu as pltpu
from jax.experimental.pallas import tpu_sc as plsc

_BOUND = np.array([[-10.0, 10.0], [-10.0, 10.0], [-5.0, 5.0]], dtype=np.float32)
_L = 16
_F = 2
_T = 2 ** 19
_B_SCALE = 1.3819
_VOXEL = 1.25
_MAX_DIS = float(np.ceil((_BOUND[:, 1] - _BOUND[:, 0]).max()))
_N_MIN = int(_MAX_DIS / _VOXEL)
_RES = [int(np.floor(_N_MIN * (_B_SCALE ** l))) for l in range(_L)]
_DENSE = [(r + 1) ** 3 <= _T for r in _RES]
_P2 = np.uint32(2654435761)
_P3 = np.uint32(805459861)

_NW = 32
_C = 32
_R = _L * 8


def _idx_wgt_body(xn_ref, idx_ref, wgt_ref):
    x = xn_ref[0, :, :]
    y = xn_ref[1, :, :]
    z = xn_ref[2, :, :]
    for l in range(_L):
        res = _RES[l]
        s = res + 1
        px = x * res
        py = y * res
        pz = z * res
        fx = jnp.floor(px)
        fy = jnp.floor(py)
        fz = jnp.floor(pz)
        wx = px - fx
        wy = py - fy
        wz = pz - fz
        vx = (1.0 - wx, wx)
        vy = (1.0 - wy, wy)
        vz = (1.0 - wz, wz)
        ix = fx.astype(jnp.int32)
        iy = fy.astype(jnp.int32)
        iz = fz.astype(jnp.int32)
        if _DENSE[l]:
            xs = (ix, ix + 1)
            ys = (iy * s, (iy + 1) * s)
            zs = (iz * (s * s), (iz + 1) * (s * s))
            yz = [(ys[j] + zs[kk] + l * _T) for kk in range(2) for j in range(2)]
        else:
            ux = (ix.astype(jnp.uint32), (ix + 1).astype(jnp.uint32))
            uy = (iy.astype(jnp.uint32) * _P2,
                  (iy + 1).astype(jnp.uint32) * _P2)
            uz = (iz.astype(jnp.uint32) * _P3,
                  (iz + 1).astype(jnp.uint32) * _P3)
            yz = [uy[j] ^ uz[kk] for kk in range(2) for j in range(2)]
        wyz = [vy[j] * vz[kk] for kk in range(2) for j in range(2)]
        for k in range(8):
            ox, oyz = k & 1, k >> 1
            if _DENSE[l]:
                idx = xs[ox] + yz[oyz]
            else:
                h = ux[ox] ^ yz[oyz]
                idx = (h & jnp.uint32(_T - 1)).astype(jnp.int32) + l * _T
            w = vx[ox] * wyz[oyz]
            r = l * 8 + k
            idx_ref[r, :, :] = idx
            wgt_ref[r, :, :] = w


def _interleave_body(s_ref, r_ref, os_ref, or_ref):
    for ref, oref in ((s_ref, os_ref), (r_ref, or_ref)):
        x = ref[...]
        for g in range(16):
            oref[:, g * 16:g * 16 + 8] = x[:, g * 8:g * 8 + 8]
            oref[:, g * 16 + 8:g * 16 + 16] = x[:, 128 + g * 8:128 + g * 8 + 8]


def _sc_encode_body(tbl_s, tbl_r, idxh, wgth, outh, idx_v, sidx_v, wgt_v,
                    rows_v, feat_v, sem):
    n = idxh.shape[1]
    pw = n // _NW
    nch = pw // _C
    wid = lax.axis_index("s") * 2 + lax.axis_index("c")
    iota = lax.iota(jnp.int32, 16)

    def chunk_body(j, carry):
        base = wid * pw + j * _C
        pltpu.sync_copy(idxh.at[:, pl.ds(base, _C)], idx_v)
        pltpu.sync_copy(wgth.at[:, pl.ds(base, _C)], wgt_v)

        def shift_body(r, c1):
            def shift_grp(g2, c2):
                iv = idx_v[r, pl.ds(g2 * 16, 16)]
                sidx_v[pl.ds(r * _C + g2 * 16, 16)] = lax.shift_right_logical(
                    iv, 3)
                return c2
            lax.fori_loop(0, _C // 16, shift_grp, 0)
            return c1
        lax.fori_loop(0, _R, shift_body, 0)

        for tbl, foff in ((tbl_s, 0), (tbl_r, 32)):
            copies = [
                pltpu.async_copy(tbl.at[sidx_v.at[pl.ds(r * _C, _C)]],
                                 rows_v.at[pl.ds(r * _C, _C)], sem)
                for r in range(_R)
            ]
            for cp in copies:
                cp.wait()

            def lvl_body(l, c3):
                def grp_body(g, c4):
                    acc = [jnp.zeros((16,), jnp.float32) for _ in range(2)]
                    for k in range(8):
                        r = l * 8 + k
                        w = wgt_v[r, pl.ds(g * 16, 16)]
                        iv = idx_v[r, pl.ds(g * 16, 16)]
                        colbase = iv & 7
                        ridx = r * _C + g * 16 + iota
                        for c in range(2):
                            vals = plsc.load_gather(rows_v,
                                                    [ridx, colbase + c * 8])
                            acc[c] = acc[c] + vals * w
                    for c in range(2):
                        feat_v[foff + l * 2 + c, pl.ds(g * 16, 16)] = acc[c]
                    return c4
                lax.fori_loop(0, _C // 16, grp_body, 0)
                return c3
            lax.fori_loop(0, _L, lvl_body, 0)
        pltpu.sync_copy(feat_v, outh.at[:, pl.ds(base, _C)])
        return carry

    lax.fori_loop(0, nch, chunk_body, 0)


@functools.lru_cache(maxsize=2)
def _make_sc_encode(n):
    mesh = plsc.VectorSubcoreMesh(core_axis_name="c", subcore_axis_name="s")
    return functools.partial(
        pl.kernel,
        out_type=jax.ShapeDtypeStruct((64, n), jnp.float32),
        mesh=mesh,
        compiler_params=pltpu.CompilerParams(
            needs_layout_passes=False, use_tc_tiling_on_sc=False),
        scratch_types=[
            pltpu.VMEM((_R, _C), jnp.int32),
            pltpu.VMEM((_R * _C,), jnp.int32),
            pltpu.VMEM((_R, _C), jnp.float32),
            pltpu.VMEM((_R * _C, 16), jnp.float32),
            pltpu.VMEM((64, _C), jnp.float32),
            pltpu.SemaphoreType.DMA,
        ],
    )(_sc_encode_body)


def _mlp_body(f_ref, ws1, ws2, wc1, wc2, wc3, rgb_ref, sdf_ref):
    dn = (((0,), (0,)), ((), ()))
    prec = lax.Precision.HIGHEST
    fs = f_ref[0:32, :]
    fr = f_ref[32:64, :]
    h = jnp.maximum(lax.dot_general(ws1[...], fs, dn, precision=prec), 0.0)
    sdf_ref[...] = lax.dot_general(ws2[...], h, dn, precision=prec)
    h1 = jnp.maximum(lax.dot_general(wc1[...], fr, dn, precision=prec), 0.0)
    h2 = jnp.maximum(lax.dot_general(wc2[...], h1, dn, precision=prec), 0.0)
    rgb_ref[...] = jax.nn.sigmoid(
        lax.dot_general(wc3[...], h2, dn, precision=prec))


def kernel(xyz, table_sdf, table_rgb, Ws1, Ws2, Wc1, Wc2, Wc3):
    n = xyz.shape[0]
    bmin = jnp.asarray(_BOUND[:, 0])
    bdis = jnp.asarray(_BOUND[:, 1] - _BOUND[:, 0])
    xn_t = ((xyz - bmin) / bdis).T

    bnc = 512
    n8 = n // 8
    idx3, wgt3 = pl.pallas_call(
        _idx_wgt_body,
        grid=(n8 // bnc,),
        in_specs=[pl.BlockSpec((3, 8, bnc), lambda i: (0, 0, i))],
        out_specs=[pl.BlockSpec((_R, 8, bnc), lambda i: (0, 0, i)),
                   pl.BlockSpec((_R, 8, bnc), lambda i: (0, 0, i))],
        out_shape=[jax.ShapeDtypeStruct((_R, 8, n8), jnp.int32),
                   jax.ShapeDtypeStruct((_R, 8, n8), jnp.float32)],
    )(xn_t.reshape(3, 8, n8))
    idx_t = idx3.reshape(_R, n)
    wgt_t = wgt3.reshape(_R, n)

    nb = _T // 128
    tp_s = (table_sdf.reshape(_L, nb, 128, _F).transpose(0, 1, 3, 2)
            .reshape(_L * nb, 256))
    tp_r = (table_rgb.reshape(_L, nb, 128, _F).transpose(0, 1, 3, 2)
            .reshape(_L * nb, 256))
    br = 256
    il_s, il_r = pl.pallas_call(
        _interleave_body,
        grid=(_L * nb // br,),
        in_specs=[pl.BlockSpec((br, 256), lambda i: (i, 0)),
                  pl.BlockSpec((br, 256), lambda i: (i, 0))],
        out_specs=[pl.BlockSpec((br, 256), lambda i: (i, 0)),
                   pl.BlockSpec((br, 256), lambda i: (i, 0))],
        out_shape=[jax.ShapeDtypeStruct((_L * nb, 256), jnp.float32),
                   jax.ShapeDtypeStruct((_L * nb, 256), jnp.float32)],
    )(tp_s, tp_r)
    tbl_s = il_s.reshape(_L * _T // 8, 16)
    tbl_r = il_r.reshape(_L * _T // 8, 16)
    feats = _make_sc_encode(n)(tbl_s, tbl_r, idx_t, wgt_t)

    bm = 4096
    rgb_t, sdf_t = pl.pallas_call(
        _mlp_body,
        grid=(n // bm,),
        in_specs=[pl.BlockSpec((64, bm), lambda i: (0, i)),
                  pl.BlockSpec(Ws1.shape, lambda i: (0, 0)),
                  pl.BlockSpec(Ws2.shape, lambda i: (0, 0)),
                  pl.BlockSpec(Wc1.shape, lambda i: (0, 0)),
                  pl.BlockSpec(Wc2.shape, lambda i: (0, 0)),
                  pl.BlockSpec(Wc3.shape, lambda i: (0, 0))],
        out_specs=[pl.BlockSpec((3, bm), lambda i: (0, i)),
                   pl.BlockSpec((1, bm), lambda i: (0, i))],
        out_shape=[jax.ShapeDtypeStruct((3, n), jnp.float32),
                   jax.ShapeDtypeStruct((1, n), jnp.float32)],
    )(feats, Ws1, Ws2, Wc1, Wc2, Wc3)

    return (rgb_t.T, sdf_t[0])


# speedup vs baseline: 8.5809x; 1.1750x over previous
import functools

import numpy as np
import jax
import jax.numpy as jnp
from jax import lax
from jax.experimental import pallas as pl
from jax.experimental.pallas import tpu as pltpu
from jax.experimental.pallas import tpu_sc as plsc

_BOUND = np.array([[-10.0, 10.0], [-10.0, 10.0], [-5.0, 5.0]], dtype=np.float32)
_L = 16
_F = 2
_T = 2 ** 19
_B_SCALE = 1.3819
_VOXEL = 1.25
_MAX_DIS = float(np.ceil((_BOUND[:, 1] - _BOUND[:, 0]).max()))
_N_MIN = int(_MAX_DIS / _VOXEL)
_RES = [int(np.floor(_N_MIN * (_B_SCALE ** l))) for l in range(_L)]
_DENSE = [(r + 1) ** 3 <= _T for r in _RES]
_P2 = np.uint32(2654435761)
_P3 = np.uint32(805459861)

_NW = 32
_C = 32
_R = _L * 8


def _idx_wgt_body(xn_ref, idx_ref, wgt_ref):
    x = xn_ref[0, :, :]
    y = xn_ref[1, :, :]
    z = xn_ref[2, :, :]
    for l in range(_L):
        res = _RES[l]
        s = res + 1
        px = x * res
        py = y * res
        pz = z * res
        fx = jnp.floor(px)
        fy = jnp.floor(py)
        fz = jnp.floor(pz)
        wx = px - fx
        wy = py - fy
        wz = pz - fz
        vx = (1.0 - wx, wx)
        vy = (1.0 - wy, wy)
        vz = (1.0 - wz, wz)
        ix = fx.astype(jnp.int32)
        iy = fy.astype(jnp.int32)
        iz = fz.astype(jnp.int32)
        if _DENSE[l]:
            xs = (ix, ix + 1)
            ys = (iy * s, (iy + 1) * s)
            zs = (iz * (s * s), (iz + 1) * (s * s))
            yz = [(ys[j] + zs[kk] + l * _T) for kk in range(2) for j in range(2)]
        else:
            ux = (ix.astype(jnp.uint32), (ix + 1).astype(jnp.uint32))
            uy = (iy.astype(jnp.uint32) * _P2,
                  (iy + 1).astype(jnp.uint32) * _P2)
            uz = (iz.astype(jnp.uint32) * _P3,
                  (iz + 1).astype(jnp.uint32) * _P3)
            yz = [uy[j] ^ uz[kk] for kk in range(2) for j in range(2)]
        wyz = [vy[j] * vz[kk] for kk in range(2) for j in range(2)]
        for k in range(8):
            ox, oyz = k & 1, k >> 1
            if _DENSE[l]:
                idx = xs[ox] + yz[oyz]
            else:
                h = ux[ox] ^ yz[oyz]
                idx = (h & jnp.uint32(_T - 1)).astype(jnp.int32) + l * _T
            w = vx[ox] * wyz[oyz]
            r = l * 8 + k
            idx_ref[r, :, :] = idx
            wgt_ref[r, :, :] = w


def _interleave_body(s_ref, r_ref, os_ref, or_ref):
    for ref, oref in ((s_ref, os_ref), (r_ref, or_ref)):
        x = ref[...]
        for g in range(16):
            oref[:, g * 16:g * 16 + 8] = x[:, g * 8:g * 8 + 8]
            oref[:, g * 16 + 8:g * 16 + 16] = x[:, 128 + g * 8:128 + g * 8 + 8]


def _sc_encode_body(tbl_s, tbl_r, idxh, wgth, outh, idx_v, sidx_v, wgt_v,
                    rows_a, rows_b, feat_v, sem, sem2):
    n = idxh.shape[1]
    pw = n // _NW
    nch = pw // _C
    wid = lax.axis_index("s") * 2 + lax.axis_index("c")
    iota = lax.iota(jnp.int32, 16)

    def chunk_body(j, carry):
        base = wid * pw + j * _C
        pltpu.sync_copy(idxh.at[:, pl.ds(base, _C)], idx_v)
        pltpu.sync_copy(wgth.at[:, pl.ds(base, _C)], wgt_v)

        def shift_body(r, c1):
            def shift_grp(g2, c2):
                iv = idx_v[r, pl.ds(g2 * 16, 16)]
                sidx_v[pl.ds(r * _C + g2 * 16, 16)] = lax.shift_right_logical(
                    iv, 3)
                return c2
            lax.fori_loop(0, _C // 16, shift_grp, 0)
            return c1
        lax.fori_loop(0, _R, shift_body, 0)

        waves = ((tbl_s, 0, 0), (tbl_s, 0, 64), (tbl_r, 32, 0), (tbl_r, 32, 64))
        bufs = (rows_a, rows_b)
        sems = (sem, sem2)

        def fire(wi):
            tbl, _, r0 = waves[wi]
            buf = bufs[wi % 2]
            return [
                pltpu.async_copy(
                    tbl.at[sidx_v.at[pl.ds((r0 + r) * _C, _C)]],
                    buf.at[pl.ds(r * _C, _C)], sems[wi % 2])
                for r in range(64)
            ]

        cps = fire(0)
        for wi in range(4):
            nxt = fire(wi + 1) if wi < 3 else []
            for cp in cps:
                cp.wait()
            _, foff, r0 = waves[wi]
            buf = bufs[wi % 2]

            def lvl_body(ll, c3, foff=foff, r0=r0, buf=buf):
                l = r0 // 8 + ll
                def grp_body(g, c4):
                    acc = [jnp.zeros((16,), jnp.float32) for _ in range(2)]
                    for k in range(8):
                        r = l * 8 + k
                        w = wgt_v[r, pl.ds(g * 16, 16)]
                        iv = idx_v[r, pl.ds(g * 16, 16)]
                        colbase = iv & 7
                        ridx = (r - r0) * _C + g * 16 + iota
                        for c in range(2):
                            vals = plsc.load_gather(buf,
                                                    [ridx, colbase + c * 8])
                            acc[c] = acc[c] + vals * w
                    for c in range(2):
                        feat_v[foff + l * 2 + c, pl.ds(g * 16, 16)] = acc[c]
                    return c4
                lax.fori_loop(0, _C // 16, grp_body, 0)
                return c3
            lax.fori_loop(0, 8, lvl_body, 0)
            cps = nxt
        pltpu.sync_copy(feat_v, outh.at[:, pl.ds(base, _C)])
        return carry

    lax.fori_loop(0, nch, chunk_body, 0)


@functools.lru_cache(maxsize=2)
def _make_sc_encode(n):
    mesh = plsc.VectorSubcoreMesh(core_axis_name="c", subcore_axis_name="s")
    return functools.partial(
        pl.kernel,
        out_type=jax.ShapeDtypeStruct((64, n), jnp.float32),
        mesh=mesh,
        compiler_params=pltpu.CompilerParams(
            needs_layout_passes=False, use_tc_tiling_on_sc=False),
        scratch_types=[
            pltpu.VMEM((_R, _C), jnp.int32),
            pltpu.VMEM((_R * _C,), jnp.int32),
            pltpu.VMEM((_R, _C), jnp.float32),
            pltpu.VMEM((64 * _C, 16), jnp.float32),
            pltpu.VMEM((64 * _C, 16), jnp.float32),
            pltpu.VMEM((64, _C), jnp.float32),
            pltpu.SemaphoreType.DMA,
            pltpu.SemaphoreType.DMA,
        ],
    )(_sc_encode_body)


def _mlp_body(f_ref, ws1, ws2, wc1, wc2, wc3, rgb_ref, sdf_ref):
    dn = (((0,), (0,)), ((), ()))
    prec = lax.Precision.HIGHEST
    fs = f_ref[0:32, :]
    fr = f_ref[32:64, :]
    h = jnp.maximum(lax.dot_general(ws1[...], fs, dn, precision=prec), 0.0)
    sdf_ref[...] = lax.dot_general(ws2[...], h, dn, precision=prec)
    h1 = jnp.maximum(lax.dot_general(wc1[...], fr, dn, precision=prec), 0.0)
    h2 = jnp.maximum(lax.dot_general(wc2[...], h1, dn, precision=prec), 0.0)
    rgb_ref[...] = jax.nn.sigmoid(
        lax.dot_general(wc3[...], h2, dn, precision=prec))


def kernel(xyz, table_sdf, table_rgb, Ws1, Ws2, Wc1, Wc2, Wc3):
    n = xyz.shape[0]
    bmin = jnp.asarray(_BOUND[:, 0])
    bdis = jnp.asarray(_BOUND[:, 1] - _BOUND[:, 0])
    xn_t = ((xyz - bmin) / bdis).T

    bnc = 512
    n8 = n // 8
    idx3, wgt3 = pl.pallas_call(
        _idx_wgt_body,
        grid=(n8 // bnc,),
        in_specs=[pl.BlockSpec((3, 8, bnc), lambda i: (0, 0, i))],
        out_specs=[pl.BlockSpec((_R, 8, bnc), lambda i: (0, 0, i)),
                   pl.BlockSpec((_R, 8, bnc), lambda i: (0, 0, i))],
        out_shape=[jax.ShapeDtypeStruct((_R, 8, n8), jnp.int32),
                   jax.ShapeDtypeStruct((_R, 8, n8), jnp.float32)],
    )(xn_t.reshape(3, 8, n8))
    idx_t = idx3.reshape(_R, n)
    wgt_t = wgt3.reshape(_R, n)

    nb = _T // 128
    tp_s = (table_sdf.reshape(_L, nb, 128, _F).transpose(0, 1, 3, 2)
            .reshape(_L * nb, 256))
    tp_r = (table_rgb.reshape(_L, nb, 128, _F).transpose(0, 1, 3, 2)
            .reshape(_L * nb, 256))
    br = 256
    il_s, il_r = pl.pallas_call(
        _interleave_body,
        grid=(_L * nb // br,),
        in_specs=[pl.BlockSpec((br, 256), lambda i: (i, 0)),
                  pl.BlockSpec((br, 256), lambda i: (i, 0))],
        out_specs=[pl.BlockSpec((br, 256), lambda i: (i, 0)),
                   pl.BlockSpec((br, 256), lambda i: (i, 0))],
        out_shape=[jax.ShapeDtypeStruct((_L * nb, 256), jnp.float32),
                   jax.ShapeDtypeStruct((_L * nb, 256), jnp.float32)],
    )(tp_s, tp_r)
    tbl_s = il_s.reshape(_L * _T // 8, 16)
    tbl_r = il_r.reshape(_L * _T // 8, 16)
    feats = _make_sc_encode(n)(tbl_s, tbl_r, idx_t, wgt_t)

    bm = 4096
    rgb_t, sdf_t = pl.pallas_call(
        _mlp_body,
        grid=(n // bm,),
        in_specs=[pl.BlockSpec((64, bm), lambda i: (0, i)),
                  pl.BlockSpec(Ws1.shape, lambda i: (0, 0)),
                  pl.BlockSpec(Ws2.shape, lambda i: (0, 0)),
                  pl.BlockSpec(Wc1.shape, lambda i: (0, 0)),
                  pl.BlockSpec(Wc2.shape, lambda i: (0, 0)),
                  pl.BlockSpec(Wc3.shape, lambda i: (0, 0))],
        out_specs=[pl.BlockSpec((3, bm), lambda i: (0, i)),
                   pl.BlockSpec((1, bm), lambda i: (0, i))],
        out_shape=[jax.ShapeDtypeStruct((3, n), jnp.float32),
                   jax.ShapeDtypeStruct((1, n), jnp.float32)],
    )(feats, Ws1, Ws2, Wc1, Wc2, Wc3)

    return (rgb_t.T, sdf_t[0])
